# Initial kernel scaffold; baseline (speedup 1.0000x reference)
import jax
import jax.numpy as jnp
from jax.experimental import pallas as pl


def kernel(z_e, W):
    raise NotImplementedError("write your pallas kernel here")


# speedup vs baseline: 1.0343x; 1.0343x over previous
import functools

import jax
import jax.numpy as jnp
from jax.experimental import pallas as pl
from jax.experimental.pallas import tpu as pltpu

_CCOST = 0.25
_BLK = 512


def _vq_body(z_ref, w_ref, z2_ref, w2_ref,
             enc_ref, zq_ref, loss_ref, perp_ref,
             counts_ref, acc_ref, *, nsteps, n_total, n_codes):
    i = pl.program_id(0)
    z = z_ref[...]
    w = w_ref[...]
    zw = jax.lax.dot_general(z, w, (((1,), (1,)), ((), ())),
                             preferred_element_type=jnp.float32)
    d = z2_ref[...] - 2.0 * zw + w2_ref[...]
    dmin = jnp.min(d, axis=1, keepdims=True)
    iota = jax.lax.broadcasted_iota(jnp.int32, d.shape, 1)
    idx = jnp.min(jnp.where(d == dmin, iota, jnp.int32(n_codes)), axis=1)
    onehot = (iota == idx[:, None]).astype(jnp.float32)
    enc_ref[...] = onehot
    zq = jax.lax.dot_general(onehot, w, (((1,), (0,)), ((), ())),
                             precision=jax.lax.Precision.HIGHEST,
                             preferred_element_type=jnp.float32)
    zq_ref[...] = z + (zq - z)
    diff = zq - z
    sq = jnp.sum(diff * diff)
    cnt = jnp.sum(onehot, axis=0, keepdims=True)

    @pl.when(i == 0)
    def _init():
        acc_ref[0, 0] = 0.0
        counts_ref[...] = jnp.zeros_like(counts_ref)

    acc_ref[0, 0] += sq
    counts_ref[...] += cnt

    @pl.when(i == nsteps - 1)
    def _finalize():
        mean_sq = acc_ref[0, 0] / jnp.float32(n_total * z.shape[1])
        loss_ref[...] = jnp.reshape((1.0 + _CCOST) * mean_sq, (1, 1))
        e_mean = counts_ref[...] / jnp.float32(n_total)
        ent = jnp.sum(e_mean * jnp.log(e_mean + 1e-10))
        perp_ref[...] = jnp.reshape(jnp.exp(-ent), (1, 1))


def kernel(z_e, W):
    B, C, H, Wd = z_e.shape
    K, D = W.shape
    N = B * H * Wd
    nsteps = N // _BLK
    zf = jnp.transpose(z_e, (0, 2, 3, 1)).reshape(N, D)
    z2 = jnp.sum(zf ** 2, axis=1, keepdims=True)
    w2 = jnp.sum(W ** 2, axis=1)[None, :]
    enc, zq_flat, loss, perp = pl.pallas_call(
        functools.partial(_vq_body, nsteps=nsteps, n_total=N, n_codes=K),
        grid=(nsteps,),
        in_specs=[
            pl.BlockSpec((_BLK, D), lambda i: (i, 0)),
            pl.BlockSpec((K, D), lambda i: (0, 0)),
            pl.BlockSpec((_BLK, 1), lambda i: (i, 0)),
            pl.BlockSpec((1, K), lambda i: (0, 0)),
        ],
        out_specs=[
            pl.BlockSpec((_BLK, K), lambda i: (i, 0)),
            pl.BlockSpec((_BLK, D), lambda i: (i, 0)),
            pl.BlockSpec((1, 1), lambda i: (0, 0)),
            pl.BlockSpec((1, 1), lambda i: (0, 0)),
        ],
        out_shape=[
            jax.ShapeDtypeStruct((N, K), jnp.float32),
            jax.ShapeDtypeStruct((N, D), jnp.float32),
            jax.ShapeDtypeStruct((1, 1), jnp.float32),
            jax.ShapeDtypeStruct((1, 1), jnp.float32),
        ],
        scratch_shapes=[
            pltpu.VMEM((1, K), jnp.float32),
            pltpu.SMEM((1, 1), jnp.float32),
        ],
    )(zf, W, z2, w2)
    z_q = jnp.transpose(zq_flat.reshape(B, H, Wd, D), (0, 3, 1, 2))
    return (z_q, loss[0, 0], perp[0, 0], enc)


# speedup vs baseline: 1.0618x; 1.0266x over previous
import functools

import jax
import jax.numpy as jnp
from jax.experimental import pallas as pl
from jax.experimental.pallas import tpu as pltpu

_CCOST = 0.25
_BLK = 512


def _vq_body(z_ref, w_ref, z2_ref, w2_ref,
             enc_ref, zq_ref, loss_ref, perp_ref,
             counts_ref, acc_ref, *, nsteps, n_total, n_codes):
    i = pl.program_id(0)
    z = z_ref[...]
    w = w_ref[...]
    zw2 = jax.lax.dot_general(z + z, w, (((1,), (1,)), ((), ())),
                              preferred_element_type=jnp.float32)
    d = z2_ref[...] - zw2 + w2_ref[...]
    dmin = jnp.min(d, axis=1, keepdims=True)
    iota = jax.lax.broadcasted_iota(jnp.int32, d.shape, 1)
    idx = jnp.min(jnp.where(d == dmin, iota, jnp.int32(n_codes)), axis=1)
    onehot = (iota == idx[:, None]).astype(jnp.float32)
    enc_ref[...] = onehot
    zq = jax.lax.dot_general(onehot, w, (((1,), (0,)), ((), ())),
                             precision=jax.lax.Precision.HIGHEST,
                             preferred_element_type=jnp.float32)
    zq_ref[...] = z + (zq - z)
    sq = jnp.sum(dmin)
    cnt = jnp.sum(onehot, axis=0, keepdims=True)

    @pl.when(i == 0)
    def _init():
        acc_ref[0, 0] = 0.0
        counts_ref[...] = jnp.zeros_like(counts_ref)

    acc_ref[0, 0] += sq
    counts_ref[...] += cnt

    @pl.when(i == nsteps - 1)
    def _finalize():
        mean_sq = acc_ref[0, 0] / jnp.float32(n_total * z.shape[1])
        loss_ref[...] = jnp.reshape((1.0 + _CCOST) * mean_sq, (1, 1))
        e_mean = counts_ref[...] / jnp.float32(n_total)
        ent = jnp.sum(e_mean * jnp.log(e_mean + 1e-10))
        perp_ref[...] = jnp.reshape(jnp.exp(-ent), (1, 1))


def kernel(z_e, W):
    B, C, H, Wd = z_e.shape
    K, D = W.shape
    N = B * H * Wd
    nsteps = N // _BLK
    zf = jnp.transpose(z_e, (0, 2, 3, 1)).reshape(N, D)
    z2 = jnp.sum(zf ** 2, axis=1, keepdims=True)
    w2 = jnp.sum(W ** 2, axis=1)[None, :]
    enc, zq_flat, loss, perp = pl.pallas_call(
        functools.partial(_vq_body, nsteps=nsteps, n_total=N, n_codes=K),
        grid=(nsteps,),
        in_specs=[
            pl.BlockSpec((_BLK, D), lambda i: (i, 0)),
            pl.BlockSpec((K, D), lambda i: (0, 0)),
            pl.BlockSpec((_BLK, 1), lambda i: (i, 0)),
            pl.BlockSpec((1, K), lambda i: (0, 0)),
        ],
        out_specs=[
            pl.BlockSpec((_BLK, K), lambda i: (i, 0)),
            pl.BlockSpec((_BLK, D), lambda i: (i, 0)),
            pl.BlockSpec((1, 1), lambda i: (0, 0)),
            pl.BlockSpec((1, 1), lambda i: (0, 0)),
        ],
        out_shape=[
            jax.ShapeDtypeStruct((N, K), jnp.float32),
            jax.ShapeDtypeStruct((N, D), jnp.float32),
            jax.ShapeDtypeStruct((1, 1), jnp.float32),
            jax.ShapeDtypeStruct((1, 1), jnp.float32),
        ],
        scratch_shapes=[
            pltpu.VMEM((1, K), jnp.float32),
            pltpu.SMEM((1, 1), jnp.float32),
        ],
    )(zf, W, z2, w2)
    z_q = jnp.transpose(zq_flat.reshape(B, H, Wd, D), (0, 3, 1, 2))
    return (z_q, loss[0, 0], perp[0, 0], enc)


# speedup vs baseline: 1.3247x; 1.2476x over previous
import functools

import jax
import jax.numpy as jnp
from jax.experimental import pallas as pl
from jax.experimental.pallas import tpu as pltpu
from jax.experimental.pallas import tpu_sc as plsc

_CCOST = 0.25
_BLK = 512
_GW = 128


def _vq_body(z_ref, w_ref, z2_ref, w2_ref,
             enc_ref, idx_ref, loss_ref, perp_ref,
             counts_ref, acc_ref, *, nsteps, n_total, n_codes):
    i = pl.program_id(0)
    z = z_ref[...]
    w = w_ref[...]
    zw2 = jax.lax.dot_general(z + z, w, (((1,), (1,)), ((), ())),
                              preferred_element_type=jnp.float32)
    d = z2_ref[...] - zw2 + w2_ref[...]
    dmin = jnp.min(d, axis=1, keepdims=True)
    iota = jax.lax.broadcasted_iota(jnp.int32, d.shape, 1)
    idx = jnp.min(jnp.where(d == dmin, iota, jnp.int32(n_codes)), axis=1)
    onehot = (iota == idx[:, None]).astype(jnp.float32)
    enc_ref[...] = onehot
    idx_ref[...] = idx[:, None]
    sq = jnp.sum(dmin)
    cnt = jnp.sum(onehot, axis=0, keepdims=True)

    @pl.when(i == 0)
    def _init():
        acc_ref[0, 0] = 0.0
        counts_ref[...] = jnp.zeros_like(counts_ref)

    acc_ref[0, 0] += sq
    counts_ref[...] += cnt

    @pl.when(i == nsteps - 1)
    def _finalize():
        mean_sq = acc_ref[0, 0] / jnp.float32(n_total * z.shape[1])
        loss_ref[...] = jnp.reshape((1.0 + _CCOST) * mean_sq, (1, 1))
        e_mean = counts_ref[...] / jnp.float32(n_total)
        ent = jnp.sum(e_mean * jnp.log(e_mean + 1e-10))
        perp_ref[...] = jnp.reshape(jnp.exp(-ent), (1, 1))


def _sc_gather_rows(W, idx_row, n, d):
    mesh = plsc.VectorSubcoreMesh(core_axis_name="core",
                                  subcore_axis_name="subcore")

    @pl.kernel(out_type=jax.ShapeDtypeStruct((n, d), jnp.float32), mesh=mesh)
    def sc_kernel(w_hbm, i_hbm, o_hbm):
        def body(i_vmem, o_vmem):
            pltpu.sync_copy(w_hbm.at[i_vmem.at[0]], o_vmem)

        pltpu.emit_pipeline(
            body,
            grid=(n // _GW,),
            in_specs=[pl.BlockSpec((1, _GW), lambda i: (0, i))],
            out_specs=[pl.BlockSpec((_GW, d), lambda i: (i, 0))],
            core_axis_name=("core", "subcore"),
            dimension_semantics=(pltpu.PARALLEL,),
        )(i_hbm, o_hbm)

    return sc_kernel(W, idx_row)


def kernel(z_e, W):
    B, C, H, Wd = z_e.shape
    K, D = W.shape
    N = B * H * Wd
    nsteps = N // _BLK
    zf = jnp.transpose(z_e, (0, 2, 3, 1)).reshape(N, D)
    z2 = jnp.sum(zf ** 2, axis=1, keepdims=True)
    w2 = jnp.sum(W ** 2, axis=1)[None, :]
    enc, idx, loss, perp = pl.pallas_call(
        functools.partial(_vq_body, nsteps=nsteps, n_total=N, n_codes=K),
        grid=(nsteps,),
        in_specs=[
            pl.BlockSpec((_BLK, D), lambda i: (i, 0)),
            pl.BlockSpec((K, D), lambda i: (0, 0)),
            pl.BlockSpec((_BLK, 1), lambda i: (i, 0)),
            pl.BlockSpec((1, K), lambda i: (0, 0)),
        ],
        out_specs=[
            pl.BlockSpec((_BLK, K), lambda i: (i, 0)),
            pl.BlockSpec((_BLK, 1), lambda i: (i, 0)),
            pl.BlockSpec((1, 1), lambda i: (0, 0)),
            pl.BlockSpec((1, 1), lambda i: (0, 0)),
        ],
        out_shape=[
            jax.ShapeDtypeStruct((N, K), jnp.float32),
            jax.ShapeDtypeStruct((N, 1), jnp.int32),
            jax.ShapeDtypeStruct((1, 1), jnp.float32),
            jax.ShapeDtypeStruct((1, 1), jnp.float32),
        ],
        scratch_shapes=[
            pltpu.VMEM((1, K), jnp.float32),
            pltpu.SMEM((1, 1), jnp.float32),
        ],
    )(zf, W, z2, w2)
    W_pad = jnp.concatenate([W, jnp.zeros_like(W)], axis=1)
    zq_pad = _sc_gather_rows(W_pad, idx.reshape(1, N), N, 2 * D)
    z_q = jnp.transpose(zq_pad.reshape(B, H, Wd, 2 * D)[..., :D], (0, 3, 1, 2))
    return (z_q, loss[0, 0], perp[0, 0], enc)


# speedup vs baseline: 1.4622x; 1.1038x over previous
import functools

import jax
import jax.numpy as jnp
from jax.experimental import pallas as pl
from jax.experimental.pallas import tpu as pltpu
from jax.experimental.pallas import tpu_sc as plsc

_CCOST = 0.25
_BLK = 1024
_GW = 128


def _vq_body(z_ref, w_ref, z2_ref, w2_ref,
             enc_ref, idx_ref, loss_ref, perp_ref,
             counts_ref, acc_ref, *, nsteps, n_total, n_codes):
    i = pl.program_id(0)
    z = z_ref[...]
    w = w_ref[...]
    zw2 = jax.lax.dot_general(z + z, w, (((1,), (1,)), ((), ())),
                              preferred_element_type=jnp.float32)
    d = z2_ref[...] - zw2 + w2_ref[...]
    dmin = jnp.min(d, axis=1, keepdims=True)
    iota = jax.lax.broadcasted_iota(jnp.int32, d.shape, 1)
    idx = jnp.min(jnp.where(d == dmin, iota, jnp.int32(n_codes)), axis=1)
    onehot = (iota == idx[:, None]).astype(jnp.float32)
    enc_ref[...] = onehot
    idx_ref[...] = idx[:, None]
    sq = jnp.sum(dmin)
    cnt = jnp.sum(onehot, axis=0, keepdims=True)

    @pl.when(i == 0)
    def _init():
        acc_ref[0, 0] = 0.0
        counts_ref[...] = jnp.zeros_like(counts_ref)

    acc_ref[0, 0] += sq
    counts_ref[...] += cnt

    @pl.when(i == nsteps - 1)
    def _finalize():
        mean_sq = acc_ref[0, 0] / jnp.float32(n_total * z.shape[1])
        loss_ref[...] = jnp.reshape((1.0 + _CCOST) * mean_sq, (1, 1))
        e_mean = counts_ref[...] / jnp.float32(n_total)
        ent = jnp.sum(e_mean * jnp.log(e_mean + 1e-10))
        perp_ref[...] = jnp.reshape(jnp.exp(-ent), (1, 1))


def _sc_gather_rows(W, idx_row, n, d):
    mesh = plsc.VectorSubcoreMesh(core_axis_name="core",
                                  subcore_axis_name="subcore")

    @pl.kernel(out_type=jax.ShapeDtypeStruct((n, d), jnp.float32), mesh=mesh)
    def sc_kernel(w_hbm, i_hbm, o_hbm):
        def body(i_vmem, o_vmem):
            pltpu.sync_copy(w_hbm.at[i_vmem.at[0]], o_vmem)

        pltpu.emit_pipeline(
            body,
            grid=(n // _GW,),
            in_specs=[pl.BlockSpec((1, _GW), lambda i: (0, i))],
            out_specs=[pl.BlockSpec((_GW, d), lambda i: (i, 0))],
            core_axis_name=("core", "subcore"),
            dimension_semantics=(pltpu.PARALLEL,),
        )(i_hbm, o_hbm)

    return sc_kernel(W, idx_row)


def kernel(z_e, W):
    B, C, H, Wd = z_e.shape
    K, D = W.shape
    N = B * H * Wd
    nsteps = N // _BLK
    zf = jnp.transpose(z_e, (0, 2, 3, 1)).reshape(N, D)
    z2 = jnp.sum(zf ** 2, axis=1, keepdims=True)
    w2 = jnp.sum(W ** 2, axis=1)[None, :]
    enc, idx, loss, perp = pl.pallas_call(
        functools.partial(_vq_body, nsteps=nsteps, n_total=N, n_codes=K),
        grid=(nsteps,),
        in_specs=[
            pl.BlockSpec((_BLK, D), lambda i: (i, 0)),
            pl.BlockSpec((K, D), lambda i: (0, 0)),
            pl.BlockSpec((_BLK, 1), lambda i: (i, 0)),
            pl.BlockSpec((1, K), lambda i: (0, 0)),
        ],
        out_specs=[
            pl.BlockSpec((_BLK, K), lambda i: (i, 0)),
            pl.BlockSpec((_BLK, 1), lambda i: (i, 0)),
            pl.BlockSpec((1, 1), lambda i: (0, 0)),
            pl.BlockSpec((1, 1), lambda i: (0, 0)),
        ],
        out_shape=[
            jax.ShapeDtypeStruct((N, K), jnp.float32),
            jax.ShapeDtypeStruct((N, 1), jnp.int32),
            jax.ShapeDtypeStruct((1, 1), jnp.float32),
            jax.ShapeDtypeStruct((1, 1), jnp.float32),
        ],
        scratch_shapes=[
            pltpu.VMEM((1, K), jnp.float32),
            pltpu.SMEM((1, 1), jnp.float32),
        ],
    )(zf, W, z2, w2)
    W_pad = jnp.concatenate([W, jnp.zeros_like(W)], axis=1)
    zq_pad = _sc_gather_rows(W_pad, idx.reshape(1, N), N, 2 * D)
    z_q = jnp.transpose(zq_pad.reshape(B, H, Wd, 2 * D)[..., :D], (0, 3, 1, 2))
    return (z_q, loss[0, 0], perp[0, 0], enc)


# speedup vs baseline: 1.5548x; 1.0633x over previous
import functools

import jax
import jax.numpy as jnp
from jax.experimental import pallas as pl
from jax.experimental.pallas import tpu as pltpu
from jax.experimental.pallas import tpu_sc as plsc

_CCOST = 0.25
_BLK = 2048
_GW = 128


def _vq_body(z_ref, w_ref, z2_ref, w2_ref,
             enc_ref, idx_ref, loss_ref, perp_ref,
             counts_ref, acc_ref, *, nsteps, n_total, n_codes):
    i = pl.program_id(0)
    z = z_ref[...]
    w = w_ref[...]
    zw2 = jax.lax.dot_general(z + z, w, (((1,), (1,)), ((), ())),
                              preferred_element_type=jnp.float32)
    d = z2_ref[...] - zw2 + w2_ref[...]
    dmin = jnp.min(d, axis=1, keepdims=True)
    iota = jax.lax.broadcasted_iota(jnp.int32, d.shape, 1)
    idx = jnp.min(jnp.where(d == dmin, iota, jnp.int32(n_codes)), axis=1)
    onehot = (iota == idx[:, None]).astype(jnp.float32)
    enc_ref[...] = onehot
    idx_ref[...] = idx[:, None]
    sq = jnp.sum(dmin)
    cnt = jnp.sum(onehot, axis=0, keepdims=True)

    @pl.when(i == 0)
    def _init():
        acc_ref[0, 0] = 0.0
        counts_ref[...] = jnp.zeros_like(counts_ref)

    acc_ref[0, 0] += sq
    counts_ref[...] += cnt

    @pl.when(i == nsteps - 1)
    def _finalize():
        mean_sq = acc_ref[0, 0] / jnp.float32(n_total * z.shape[1])
        loss_ref[...] = jnp.reshape((1.0 + _CCOST) * mean_sq, (1, 1))
        e_mean = counts_ref[...] / jnp.float32(n_total)
        ent = jnp.sum(e_mean * jnp.log(e_mean + 1e-10))
        perp_ref[...] = jnp.reshape(jnp.exp(-ent), (1, 1))


def _sc_gather_rows(W, idx_row, n, d):
    mesh = plsc.VectorSubcoreMesh(core_axis_name="core",
                                  subcore_axis_name="subcore")

    @pl.kernel(out_type=jax.ShapeDtypeStruct((n, d), jnp.float32), mesh=mesh)
    def sc_kernel(w_hbm, i_hbm, o_hbm):
        def body(i_vmem, o_vmem):
            pltpu.sync_copy(w_hbm.at[i_vmem.at[0]], o_vmem)

        pltpu.emit_pipeline(
            body,
            grid=(n // _GW,),
            in_specs=[pl.BlockSpec((1, _GW), lambda i: (0, i))],
            out_specs=[pl.BlockSpec((_GW, d), lambda i: (i, 0))],
            core_axis_name=("core", "subcore"),
            dimension_semantics=(pltpu.PARALLEL,),
        )(i_hbm, o_hbm)

    return sc_kernel(W, idx_row)


def kernel(z_e, W):
    B, C, H, Wd = z_e.shape
    K, D = W.shape
    N = B * H * Wd
    nsteps = N // _BLK
    zf = jnp.transpose(z_e, (0, 2, 3, 1)).reshape(N, D)
    z2 = jnp.sum(zf ** 2, axis=1, keepdims=True)
    w2 = jnp.sum(W ** 2, axis=1)[None, :]
    enc, idx, loss, perp = pl.pallas_call(
        functools.partial(_vq_body, nsteps=nsteps, n_total=N, n_codes=K),
        grid=(nsteps,),
        in_specs=[
            pl.BlockSpec((_BLK, D), lambda i: (i, 0)),
            pl.BlockSpec((K, D), lambda i: (0, 0)),
            pl.BlockSpec((_BLK, 1), lambda i: (i, 0)),
            pl.BlockSpec((1, K), lambda i: (0, 0)),
        ],
        out_specs=[
            pl.BlockSpec((_BLK, K), lambda i: (i, 0)),
            pl.BlockSpec((_BLK, 1), lambda i: (i, 0)),
            pl.BlockSpec((1, 1), lambda i: (0, 0)),
            pl.BlockSpec((1, 1), lambda i: (0, 0)),
        ],
        out_shape=[
            jax.ShapeDtypeStruct((N, K), jnp.float32),
            jax.ShapeDtypeStruct((N, 1), jnp.int32),
            jax.ShapeDtypeStruct((1, 1), jnp.float32),
            jax.ShapeDtypeStruct((1, 1), jnp.float32),
        ],
        scratch_shapes=[
            pltpu.VMEM((1, K), jnp.float32),
            pltpu.SMEM((1, 1), jnp.float32),
        ],
    )(zf, W, z2, w2)
    W_pad = jnp.concatenate([W, jnp.zeros_like(W)], axis=1)
    zq_pad = _sc_gather_rows(W_pad, idx.reshape(1, N), N, 2 * D)
    z_q = jnp.transpose(zq_pad.reshape(B, H, Wd, 2 * D)[..., :D], (0, 3, 1, 2))
    return (z_q, loss[0, 0], perp[0, 0], enc)
